# baseline (speedup 1.0000x reference)
import functools

import jax
import jax.numpy as jnp
from jax import lax
from jax.experimental import pallas as pl
from jax.experimental.pallas import tpu as pltpu
from jax.experimental.pallas import tpu_sc as plsc

MIN_N = 4
HID = 32
FEAT = 64
L = 16
NEG = -3.0e38


def _sc_call(pts_flat, ppre, nlen, W1b, b1b, W2b, b2b, Whead, defaults):
    P = pts_flat.shape[0]
    NW = 32
    PW = P // NW
    CH = min(PW, 64)
    NCHUNK = PW // CH
    mesh = plsc.VectorSubcoreMesh(core_axis_name="c", subcore_axis_name="s")

    @functools.partial(
        pl.kernel,
        mesh=mesh,
        compiler_params=pltpu.CompilerParams(
            needs_layout_passes=False, use_tc_tiling_on_sc=False
        ),
        out_type=jax.ShapeDtypeStruct((P, L), jnp.float32),
        scratch_types=[
            pltpu.VMEM((CH, 3 * 256), jnp.float32),
            pltpu.VMEM((CH, L), jnp.float32),
            pltpu.VMEM((CH,), jnp.int32),
            pltpu.VMEM((3, HID, L), jnp.float32),
            pltpu.VMEM((HID, L), jnp.float32),
            pltpu.VMEM((HID, FEAT, L), jnp.float32),
            pltpu.VMEM((FEAT, L), jnp.float32),
            pltpu.VMEM((2 * FEAT, L), jnp.float32),
            pltpu.VMEM((L,), jnp.float32),
            pltpu.VMEM((FEAT, L), jnp.float32),
            pltpu.VMEM((FEAT, L), jnp.float32),
            pltpu.VMEM((CH, L), jnp.float32),
        ],
    )
    def body(pts_h, ppre_h, nlen_h, W1b_h, b1b_h, W2b_h, b2b_h, Wh_h, df_h,
             out_h, pts_v, ppre_v, nlen_v, W1v, b1v, W2v, b2v, Whv, dfv,
             amax, asum, outv):
        wid = lax.axis_index("s") * 2 + lax.axis_index("c")
        pltpu.sync_copy(W1b_h, W1v)
        pltpu.sync_copy(b1b_h, b1v)
        pltpu.sync_copy(W2b_h, W2v)
        pltpu.sync_copy(b2b_h, b2v)
        pltpu.sync_copy(Wh_h, Whv)
        pltpu.sync_copy(df_h, dfv)

        iota = lax.iota(jnp.int32, L)

        def do_proposal(k, _):
            ksp = jnp.full((L,), k, jnp.int32)
            nvec = plsc.load_gather(nlen_v, [ksp])
            n_s = jnp.max(nvec)

            def splat(d):
                return plsc.load_gather(ppre_v, [ksp, jnp.full((L,), d, jnp.int32)])

            cx, cy, cz = splat(0), splat(1), splat(2)
            ivx, ivy, ivz = splat(3), splat(4), splat(5)
            qf, qp1, rf = splat(6), splat(7), splat(8)

            outv[k] = dfv[...]

            @pl.when(n_s >= MIN_N)
            def _():
                def init_acc(o, _):
                    amax[o] = jnp.full((L,), NEG, jnp.float32)
                    asum[o] = jnp.zeros((L,), jnp.float32)
                    return ()
                lax.fori_loop(0, FEAT, init_acc, ())

                ngroups = lax.shift_right_logical(n_s + (L - 1), 4)

                def do_group(g, _):
                    jvec = g * L + iota
                    mask = jvec < nvec
                    jc = jnp.minimum(jvec, 255) * 3
                    x = plsc.load_gather(pts_v, [ksp, jc])
                    y = plsc.load_gather(pts_v, [ksp, jc + 1])
                    z = plsc.load_gather(pts_v, [ksp, jc + 2])
                    x = (x - cx) * ivx
                    y = (y - cy) * ivy
                    z = (z - cz) * ivz
                    jf = jvec.astype(jnp.float32)
                    cw = jnp.where(mask, jnp.where(jf < rf, qp1, qf), 0.0)
                    h = []
                    for i in range(HID):
                        hv = x * W1v[0, i] + y * W1v[1, i] + z * W1v[2, i]
                        h.append(jnp.maximum(hv + b1v[i], 0.0))

                    def do_out(o, _):
                        f = b2v[o]
                        for i in range(HID):
                            f = f + h[i] * W2v[i, o]
                        fm = jnp.where(mask, f, NEG)
                        amax[o] = jnp.maximum(amax[o], fm)
                        asum[o] = asum[o] + cw * f
                        return ()
                    lax.fori_loop(0, FEAT, do_out, ())
                    return ()
                lax.fori_loop(0, ngroups, do_group, ())

                def heads(o, res):
                    smax = jnp.max(amax[o])
                    ssum = jnp.sum(asum[o])
                    return res + smax * Whv[o] + ssum * Whv[FEAT + o]
                resv = lax.fori_loop(0, FEAT, heads, dfv[...])
                outv[k] = resv
            return ()

        for cc in range(NCHUNK):
            base = wid * PW + cc * CH
            pltpu.sync_copy(pts_h.at[pl.ds(base, CH)], pts_v)
            pltpu.sync_copy(ppre_h.at[pl.ds(base, CH)], ppre_v)
            pltpu.sync_copy(nlen_h.at[pl.ds(base, CH)], nlen_v)
            lax.fori_loop(0, CH, do_proposal, ())
            pltpu.sync_copy(outv, out_h.at[pl.ds(base, CH)])

    return body(pts_flat, ppre, nlen, W1b, b1b, W2b, b2b, Whead, defaults)


def _tc_call(pts128, c4, iv4, vb, W1x, b1x, W2p, b2p, Whead):
    PPR = 32
    NB1 = PPR * 128
    P = pts128.shape[0]
    MAXN = 256
    BP = 32
    R = BP * 256 // PPR
    M = BP * MAXN
    grid = (P // BP,)

    def body(pts_ref, c_ref, iv_ref, vb_ref, W1_ref, b1_ref, W2_ref, b2_ref,
             Wh_ref, out_ref):
        c4 = c_ref[...][:, :4]
        iv4 = iv_ref[...][:, :4]
        ctile = jnp.concatenate([c4] * PPR, axis=1)
        ivtile = jnp.concatenate([iv4] * PPR, axis=1)
        RPP = 256 // PPR
        cb = jnp.broadcast_to(ctile[:, None, :], (BP, RPP, 128)).reshape(R, 128)
        ivb = jnp.broadcast_to(ivtile[:, None, :], (BP, RPP, 128)).reshape(R, 128)
        x128 = (pts_ref[...].reshape(R, 128) - cb) * ivb
        hb = jnp.maximum(
            jnp.dot(x128, W1_ref[...], preferred_element_type=jnp.float32)
            + b1_ref[...][0][None, :], 0.0
        ).astype(jnp.bfloat16)
        f = (jnp.dot(hb.reshape(M, 128), W2_ref[...],
                     preferred_element_type=jnp.float32)
             + b2_ref[...][0][None, :])
        cw3 = f[:, FEAT:FEAT + 1].reshape(BP, MAXN, 1)
        f3 = f[:, :FEAT].reshape(BP, MAXN, FEAT)
        fw = f3 * cw3
        fmax = jnp.max(f3, axis=1)
        fsum = jnp.sum(fw, axis=1)
        valid = vb_ref[...][:, 0:1] > 0.0
        fmax = jnp.where(valid, fmax, 0.0)
        fsum = jnp.where(valid, fsum, 0.0)
        feats = jnp.concatenate(
            [fmax, fsum, jnp.ones((BP, 8), jnp.float32)], axis=-1)
        out_ref[...] = jnp.dot(feats, Wh_ref[...],
                               preferred_element_type=jnp.float32)

    return pl.pallas_call(
        body,
        grid=grid,
        in_specs=[
            pl.BlockSpec((BP, MAXN * 4), lambda i: (i, 0)),
            pl.BlockSpec((BP, 8), lambda i: (i, 0)),
            pl.BlockSpec((BP, 8), lambda i: (i, 0)),
            pl.BlockSpec((BP, 8), lambda i: (i, 0)),
            pl.BlockSpec((128, NB1), lambda i: (0, 0)),
            pl.BlockSpec((8, NB1), lambda i: (0, 0)),
            pl.BlockSpec((128, 128), lambda i: (0, 0)),
            pl.BlockSpec((8, 128), lambda i: (0, 0)),
            pl.BlockSpec((2 * FEAT + 8, 8), lambda i: (0, 0)),
        ],
        out_specs=pl.BlockSpec((BP, 8), lambda i: (i, 0)),
        out_shape=jax.ShapeDtypeStruct((P, 8), jnp.float32),
    )(pts128, c4, iv4, vb, W1x, b1x, W2p, b2p, Whead)


SC_ROWS = 1024


def kernel(points, proposals, W1, b1, W2, b2, Wc, bc, Wr, br, lengths):
    P, MAXN, _ = points.shape
    n = lengths.astype(jnp.int32)
    safe = jnp.maximum(n, 1)
    q = MAXN // safe
    r = MAXN - q * safe
    center = proposals[:, :3]
    inv = 1.0 / (proposals[:, 3:6] + 1e-6)
    qf = q.astype(jnp.float32) / MAXN
    qp1 = qf + 1.0 / MAXN
    rf = r.astype(jnp.float32)

    PSC = SC_ROWS
    cls_parts, reg_parts = [], []

    if PSC > 0:
        ppre = jnp.zeros((PSC, L), jnp.float32)
        ppre = ppre.at[:, 0:3].set(center[:PSC]).at[:, 3:6].set(inv[:PSC])
        ppre = (ppre.at[:, 6].set(qf[:PSC]).at[:, 7].set(qp1[:PSC])
                .at[:, 8].set(rf[:PSC]))
        pts_flat = points[:PSC].reshape(PSC, MAXN * 3)
        W1b = jnp.broadcast_to(W1[:, :, None], (3, HID, L))
        b1b = jnp.broadcast_to(b1[:, None], (HID, L))
        W2b = jnp.broadcast_to(W2[:, :, None], (HID, FEAT, L))
        b2b = jnp.broadcast_to(b2[:, None], (FEAT, L))
        Whd = jnp.zeros((2 * FEAT, L), jnp.float32)
        Whd = Whd.at[:, 0].set(Wc[:, 0]).at[:, 1:5].set(Wr)
        defaults = jnp.zeros((L,), jnp.float32)
        defaults = defaults.at[0].set(bc[0]).at[1:5].set(br)
        out_sc = _sc_call(pts_flat, ppre, n[:PSC], W1b, b1b, W2b, b2b, Whd,
                          defaults)
        cls_parts.append(out_sc[:, :1])
        reg_parts.append(out_sc[:, 1:5])

    if PSC < P:
        jrow = jnp.arange(MAXN, dtype=jnp.int32)[None, :]
        cwtc = jnp.where(jrow < n[PSC:, None],
                         jnp.where(jrow < r[PSC:, None], qp1[PSC:, None],
                                   qf[PSC:, None]), 0.0)
        pts128 = jnp.concatenate(
            [points[PSC:], cwtc[:, :, None]], axis=-1
        ).reshape(P - PSC, MAXN * 4)
        c4 = jnp.zeros((P - PSC, 8), jnp.float32).at[:, :3].set(center[PSC:])
        iv4 = (jnp.zeros((P - PSC, 8), jnp.float32).at[:, :3].set(inv[PSC:])
               .at[:, 3].set(1.0))
        vb = jnp.zeros((P - PSC, 8), jnp.float32)
        vb = vb.at[:, 0].set((n[PSC:] >= MIN_N).astype(jnp.float32))
        W1blk = jnp.zeros((4, 128), jnp.float32).at[:3, :HID].set(W1)
        W1blk = W1blk.at[3, HID].set(-512.0).at[3, HID + 1].set(1.0)
        b1blk = jnp.zeros((128,), jnp.float32).at[:HID].set(b1)
        b1blk = b1blk.at[HID].set(1.0)
        W1x = jnp.kron(jnp.eye(32, dtype=jnp.float32), W1blk)
        b1x = jnp.broadcast_to(jnp.tile(b1blk, 32)[None, :], (8, 32 * 128))
        W2p = jnp.zeros((128, 128), jnp.float32).at[:HID, :FEAT].set(W2)
        W2p = W2p.at[HID, :FEAT].set(NEG).at[HID + 1, FEAT].set(1.0)
        W2p = W2p.astype(jnp.bfloat16)
        b2p = jnp.broadcast_to(
            jnp.zeros((128,), jnp.float32).at[:FEAT].set(b2)[None, :],
            (8, 128))
        Whead = jnp.zeros((2 * FEAT + 8, 8), jnp.float32)
        Whead = Whead.at[:FEAT, 0].set(Wc[:FEAT, 0]).at[:FEAT, 1:5].set(Wr[:FEAT])
        Whead = (Whead.at[FEAT:2 * FEAT, 0].set(Wc[FEAT:, 0])
                 .at[FEAT:2 * FEAT, 1:5].set(Wr[FEAT:]))
        Whead = Whead.at[2 * FEAT, 0].set(bc[0]).at[2 * FEAT, 1:5].set(br)
        out_tc = _tc_call(pts128, c4, iv4, vb, W1x, b1x, W2p, b2p, Whead)
        cls_parts.append(out_tc[:, :1])
        reg_parts.append(out_tc[:, 1:5])

    cls = jnp.concatenate(cls_parts, axis=0) if len(cls_parts) > 1 else cls_parts[0]
    reg = jnp.concatenate(reg_parts, axis=0) if len(reg_parts) > 1 else reg_parts[0]
    return cls, reg


# speedup vs baseline: 1.1654x; 1.1654x over previous
import functools

import jax
import jax.numpy as jnp
from jax import lax
from jax.experimental import pallas as pl
from jax.experimental.pallas import tpu as pltpu
from jax.experimental.pallas import tpu_sc as plsc

MIN_N = 4
HID = 32
FEAT = 64
L = 16
NEG = -3.0e38


def _sc_call(pts_flat, ppre, nlen, W1b, b1b, W2b, b2b, Whead, defaults):
    P = pts_flat.shape[0]
    NW = 32
    PW = P // NW
    CH = min(PW, 64)
    NCHUNK = PW // CH
    mesh = plsc.VectorSubcoreMesh(core_axis_name="c", subcore_axis_name="s")

    @functools.partial(
        pl.kernel,
        mesh=mesh,
        compiler_params=pltpu.CompilerParams(
            needs_layout_passes=False, use_tc_tiling_on_sc=False
        ),
        out_type=jax.ShapeDtypeStruct((P, L), jnp.float32),
        scratch_types=[
            pltpu.VMEM((CH, 3 * 256), jnp.float32),
            pltpu.VMEM((CH, L), jnp.float32),
            pltpu.VMEM((CH,), jnp.int32),
            pltpu.VMEM((3, HID, L), jnp.float32),
            pltpu.VMEM((HID, L), jnp.float32),
            pltpu.VMEM((HID, FEAT, L), jnp.float32),
            pltpu.VMEM((FEAT, L), jnp.float32),
            pltpu.VMEM((2 * FEAT, L), jnp.float32),
            pltpu.VMEM((L,), jnp.float32),
            pltpu.VMEM((FEAT, L), jnp.float32),
            pltpu.VMEM((FEAT, L), jnp.float32),
            pltpu.VMEM((CH, L), jnp.float32),
        ],
    )
    def body(pts_h, ppre_h, nlen_h, W1b_h, b1b_h, W2b_h, b2b_h, Wh_h, df_h,
             out_h, pts_v, ppre_v, nlen_v, W1v, b1v, W2v, b2v, Whv, dfv,
             amax, asum, outv):
        wid = lax.axis_index("s") * 2 + lax.axis_index("c")
        pltpu.sync_copy(W1b_h, W1v)
        pltpu.sync_copy(b1b_h, b1v)
        pltpu.sync_copy(W2b_h, W2v)
        pltpu.sync_copy(b2b_h, b2v)
        pltpu.sync_copy(Wh_h, Whv)
        pltpu.sync_copy(df_h, dfv)

        iota = lax.iota(jnp.int32, L)

        def do_proposal(k, _):
            ksp = jnp.full((L,), k, jnp.int32)
            nvec = plsc.load_gather(nlen_v, [ksp])
            n_s = jnp.max(nvec)

            def splat(d):
                return plsc.load_gather(ppre_v, [ksp, jnp.full((L,), d, jnp.int32)])

            cx, cy, cz = splat(0), splat(1), splat(2)
            ivx, ivy, ivz = splat(3), splat(4), splat(5)
            qf, qp1, rf = splat(6), splat(7), splat(8)

            outv[k] = dfv[...]

            @pl.when(n_s >= MIN_N)
            def _():
                def init_acc(o, _):
                    amax[o] = jnp.full((L,), NEG, jnp.float32)
                    asum[o] = jnp.zeros((L,), jnp.float32)
                    return ()
                lax.fori_loop(0, FEAT, init_acc, ())

                ngroups = lax.shift_right_logical(n_s + (L - 1), 4)

                def do_group(g, _):
                    jvec = g * L + iota
                    mask = jvec < nvec
                    jc = jnp.minimum(jvec, 255) * 3
                    x = plsc.load_gather(pts_v, [ksp, jc])
                    y = plsc.load_gather(pts_v, [ksp, jc + 1])
                    z = plsc.load_gather(pts_v, [ksp, jc + 2])
                    x = (x - cx) * ivx
                    y = (y - cy) * ivy
                    z = (z - cz) * ivz
                    jf = jvec.astype(jnp.float32)
                    cw = jnp.where(mask, jnp.where(jf < rf, qp1, qf), 0.0)
                    h = []
                    for i in range(HID):
                        hv = x * W1v[0, i] + y * W1v[1, i] + z * W1v[2, i]
                        h.append(jnp.maximum(hv + b1v[i], 0.0))

                    def do_out(o, _):
                        f = b2v[o]
                        for i in range(HID):
                            f = f + h[i] * W2v[i, o]
                        fm = jnp.where(mask, f, NEG)
                        amax[o] = jnp.maximum(amax[o], fm)
                        asum[o] = asum[o] + cw * f
                        return ()
                    lax.fori_loop(0, FEAT, do_out, ())
                    return ()
                lax.fori_loop(0, ngroups, do_group, ())

                def heads(o, res):
                    smax = jnp.max(amax[o])
                    ssum = jnp.sum(asum[o])
                    return res + smax * Whv[o] + ssum * Whv[FEAT + o]
                resv = lax.fori_loop(0, FEAT, heads, dfv[...])
                outv[k] = resv
            return ()

        for cc in range(NCHUNK):
            base = wid * PW + cc * CH
            pltpu.sync_copy(pts_h.at[pl.ds(base, CH)], pts_v)
            pltpu.sync_copy(ppre_h.at[pl.ds(base, CH)], ppre_v)
            pltpu.sync_copy(nlen_h.at[pl.ds(base, CH)], nlen_v)
            lax.fori_loop(0, CH, do_proposal, ())
            pltpu.sync_copy(outv, out_h.at[pl.ds(base, CH)])

    return body(pts_flat, ppre, nlen, W1b, b1b, W2b, b2b, Whead, defaults)


def _tc_call(pts128, c4, iv4, vb, W1x, b1x, W2p, b2p, Whead):
    PPR = 32
    NB1 = PPR * 128
    P = pts128.shape[0]
    MAXN = 256
    BP = 32
    R = BP * 256 // PPR
    M = BP * MAXN
    grid = (P // BP,)

    def body(pts_ref, c_ref, iv_ref, vb_ref, W1_ref, b1_ref, W2_ref, b2_ref,
             Wh_ref, out_ref):
        c4 = c_ref[...][:, :4]
        iv4 = iv_ref[...][:, :4]
        ctile = jnp.concatenate([c4] * PPR, axis=1)
        ivtile = jnp.concatenate([iv4] * PPR, axis=1)
        RPP = 256 // PPR
        cb = jnp.broadcast_to(ctile[:, None, :], (BP, RPP, 128)).reshape(R, 128)
        ivb = jnp.broadcast_to(ivtile[:, None, :], (BP, RPP, 128)).reshape(R, 128)
        x128 = (pts_ref[...].reshape(R, 128) - cb) * ivb
        hb = jnp.maximum(
            jnp.dot(x128, W1_ref[...], preferred_element_type=jnp.float32)
            + b1_ref[...][0][None, :], 0.0
        ).astype(jnp.bfloat16)
        f = (jnp.dot(hb.reshape(M, 128), W2_ref[...],
                     preferred_element_type=jnp.float32)
             + b2_ref[...][0][None, :])
        cw3 = f[:, FEAT:FEAT + 1].reshape(BP, MAXN, 1)
        f3 = f[:, :FEAT].reshape(BP, MAXN, FEAT)
        fw = f3 * cw3
        fmax = jnp.max(f3, axis=1)
        fsum = jnp.sum(fw, axis=1)
        valid = vb_ref[...][:, 0:1] > 0.0
        fmax = jnp.where(valid, fmax, 0.0)
        fsum = jnp.where(valid, fsum, 0.0)
        feats = jnp.concatenate(
            [fmax, fsum, jnp.ones((BP, 8), jnp.float32)], axis=-1)
        out_ref[...] = jnp.dot(feats, Wh_ref[...],
                               preferred_element_type=jnp.float32)

    return pl.pallas_call(
        body,
        grid=grid,
        in_specs=[
            pl.BlockSpec((BP, MAXN * 4), lambda i: (i, 0)),
            pl.BlockSpec((BP, 8), lambda i: (i, 0)),
            pl.BlockSpec((BP, 8), lambda i: (i, 0)),
            pl.BlockSpec((BP, 8), lambda i: (i, 0)),
            pl.BlockSpec((128, NB1), lambda i: (0, 0)),
            pl.BlockSpec((8, NB1), lambda i: (0, 0)),
            pl.BlockSpec((128, 128), lambda i: (0, 0)),
            pl.BlockSpec((8, 128), lambda i: (0, 0)),
            pl.BlockSpec((2 * FEAT + 8, 8), lambda i: (0, 0)),
        ],
        out_specs=pl.BlockSpec((BP, 8), lambda i: (i, 0)),
        out_shape=jax.ShapeDtypeStruct((P, 8), jnp.float32),
    )(pts128, c4, iv4, vb, W1x, b1x, W2p, b2p, Whead)


SC_ROWS = 512


def kernel(points, proposals, W1, b1, W2, b2, Wc, bc, Wr, br, lengths):
    P, MAXN, _ = points.shape
    n = lengths.astype(jnp.int32)
    safe = jnp.maximum(n, 1)
    q = MAXN // safe
    r = MAXN - q * safe
    center = proposals[:, :3]
    inv = 1.0 / (proposals[:, 3:6] + 1e-6)
    qf = q.astype(jnp.float32) / MAXN
    qp1 = qf + 1.0 / MAXN
    rf = r.astype(jnp.float32)

    PSC = SC_ROWS
    cls_parts, reg_parts = [], []

    if PSC > 0:
        ppre = jnp.zeros((PSC, L), jnp.float32)
        ppre = ppre.at[:, 0:3].set(center[:PSC]).at[:, 3:6].set(inv[:PSC])
        ppre = (ppre.at[:, 6].set(qf[:PSC]).at[:, 7].set(qp1[:PSC])
                .at[:, 8].set(rf[:PSC]))
        pts_flat = points[:PSC].reshape(PSC, MAXN * 3)
        W1b = jnp.broadcast_to(W1[:, :, None], (3, HID, L))
        b1b = jnp.broadcast_to(b1[:, None], (HID, L))
        W2b = jnp.broadcast_to(W2[:, :, None], (HID, FEAT, L))
        b2b = jnp.broadcast_to(b2[:, None], (FEAT, L))
        Whd = jnp.zeros((2 * FEAT, L), jnp.float32)
        Whd = Whd.at[:, 0].set(Wc[:, 0]).at[:, 1:5].set(Wr)
        defaults = jnp.zeros((L,), jnp.float32)
        defaults = defaults.at[0].set(bc[0]).at[1:5].set(br)
        out_sc = _sc_call(pts_flat, ppre, n[:PSC], W1b, b1b, W2b, b2b, Whd,
                          defaults)
        cls_parts.append(out_sc[:, :1])
        reg_parts.append(out_sc[:, 1:5])

    if PSC < P:
        jrow = jnp.arange(MAXN, dtype=jnp.int32)[None, :]
        cwtc = jnp.where(jrow < n[PSC:, None],
                         jnp.where(jrow < r[PSC:, None], qp1[PSC:, None],
                                   qf[PSC:, None]), 0.0)
        pts128 = jnp.concatenate(
            [points[PSC:], cwtc[:, :, None]], axis=-1
        ).reshape(P - PSC, MAXN * 4)
        c4 = jnp.zeros((P - PSC, 8), jnp.float32).at[:, :3].set(center[PSC:])
        iv4 = (jnp.zeros((P - PSC, 8), jnp.float32).at[:, :3].set(inv[PSC:])
               .at[:, 3].set(1.0))
        vb = jnp.zeros((P - PSC, 8), jnp.float32)
        vb = vb.at[:, 0].set((n[PSC:] >= MIN_N).astype(jnp.float32))
        W1blk = jnp.zeros((4, 128), jnp.float32).at[:3, :HID].set(W1)
        W1blk = W1blk.at[3, HID].set(-512.0).at[3, HID + 1].set(1.0)
        b1blk = jnp.zeros((128,), jnp.float32).at[:HID].set(b1)
        b1blk = b1blk.at[HID].set(1.0)
        W1x = jnp.kron(jnp.eye(32, dtype=jnp.float32), W1blk)
        b1x = jnp.broadcast_to(jnp.tile(b1blk, 32)[None, :], (8, 32 * 128))
        W2p = jnp.zeros((128, 128), jnp.float32).at[:HID, :FEAT].set(W2)
        W2p = W2p.at[HID, :FEAT].set(NEG).at[HID + 1, FEAT].set(1.0)
        W2p = W2p.astype(jnp.bfloat16)
        b2p = jnp.broadcast_to(
            jnp.zeros((128,), jnp.float32).at[:FEAT].set(b2)[None, :],
            (8, 128))
        Whead = jnp.zeros((2 * FEAT + 8, 8), jnp.float32)
        Whead = Whead.at[:FEAT, 0].set(Wc[:FEAT, 0]).at[:FEAT, 1:5].set(Wr[:FEAT])
        Whead = (Whead.at[FEAT:2 * FEAT, 0].set(Wc[FEAT:, 0])
                 .at[FEAT:2 * FEAT, 1:5].set(Wr[FEAT:]))
        Whead = Whead.at[2 * FEAT, 0].set(bc[0]).at[2 * FEAT, 1:5].set(br)
        out_tc = _tc_call(pts128, c4, iv4, vb, W1x, b1x, W2p, b2p, Whead)
        cls_parts.append(out_tc[:, :1])
        reg_parts.append(out_tc[:, 1:5])

    cls = jnp.concatenate(cls_parts, axis=0) if len(cls_parts) > 1 else cls_parts[0]
    reg = jnp.concatenate(reg_parts, axis=0) if len(reg_parts) > 1 else reg_parts[0]
    return cls, reg


# speedup vs baseline: 1.3174x; 1.1305x over previous
import functools

import jax
import jax.numpy as jnp
from jax import lax
from jax.experimental import pallas as pl
from jax.experimental.pallas import tpu as pltpu
from jax.experimental.pallas import tpu_sc as plsc

MIN_N = 4
HID = 32
FEAT = 64
L = 16
NEG = -3.0e38


def _sc_call(pts_flat, ppre, nlen, W1b, b1b, W2b, b2b, Whead, defaults):
    P = pts_flat.shape[0]
    NW = 32
    PW = P // NW
    CH = min(PW, 64)
    NCHUNK = PW // CH
    mesh = plsc.VectorSubcoreMesh(core_axis_name="c", subcore_axis_name="s")

    @functools.partial(
        pl.kernel,
        mesh=mesh,
        compiler_params=pltpu.CompilerParams(
            needs_layout_passes=False, use_tc_tiling_on_sc=False
        ),
        out_type=jax.ShapeDtypeStruct((P, L), jnp.float32),
        scratch_types=[
            pltpu.VMEM((CH, 3 * 256), jnp.float32),
            pltpu.VMEM((CH, L), jnp.float32),
            pltpu.VMEM((CH,), jnp.int32),
            pltpu.VMEM((3, HID, L), jnp.float32),
            pltpu.VMEM((HID, L), jnp.float32),
            pltpu.VMEM((HID, FEAT, L), jnp.float32),
            pltpu.VMEM((FEAT, L), jnp.float32),
            pltpu.VMEM((2 * FEAT, L), jnp.float32),
            pltpu.VMEM((L,), jnp.float32),
            pltpu.VMEM((FEAT, L), jnp.float32),
            pltpu.VMEM((FEAT, L), jnp.float32),
            pltpu.VMEM((CH, L), jnp.float32),
        ],
    )
    def body(pts_h, ppre_h, nlen_h, W1b_h, b1b_h, W2b_h, b2b_h, Wh_h, df_h,
             out_h, pts_v, ppre_v, nlen_v, W1v, b1v, W2v, b2v, Whv, dfv,
             amax, asum, outv):
        wid = lax.axis_index("s") * 2 + lax.axis_index("c")
        pltpu.sync_copy(W1b_h, W1v)
        pltpu.sync_copy(b1b_h, b1v)
        pltpu.sync_copy(W2b_h, W2v)
        pltpu.sync_copy(b2b_h, b2v)
        pltpu.sync_copy(Wh_h, Whv)
        pltpu.sync_copy(df_h, dfv)

        iota = lax.iota(jnp.int32, L)

        def do_proposal(k, _):
            ksp = jnp.full((L,), k, jnp.int32)
            nvec = plsc.load_gather(nlen_v, [ksp])
            n_s = jnp.max(nvec)

            def splat(d):
                return plsc.load_gather(ppre_v, [ksp, jnp.full((L,), d, jnp.int32)])

            cx, cy, cz = splat(0), splat(1), splat(2)
            ivx, ivy, ivz = splat(3), splat(4), splat(5)
            qf, qp1, rf = splat(6), splat(7), splat(8)

            outv[k] = dfv[...]

            @pl.when(n_s >= MIN_N)
            def _():
                def init_acc(o, _):
                    amax[o] = jnp.full((L,), NEG, jnp.float32)
                    asum[o] = jnp.zeros((L,), jnp.float32)
                    return ()
                lax.fori_loop(0, FEAT, init_acc, ())

                ngroups = lax.shift_right_logical(n_s + (L - 1), 4)

                def do_group(g, _):
                    jvec = g * L + iota
                    mask = jvec < nvec
                    jc = jnp.minimum(jvec, 255) * 3
                    x = plsc.load_gather(pts_v, [ksp, jc])
                    y = plsc.load_gather(pts_v, [ksp, jc + 1])
                    z = plsc.load_gather(pts_v, [ksp, jc + 2])
                    x = (x - cx) * ivx
                    y = (y - cy) * ivy
                    z = (z - cz) * ivz
                    jf = jvec.astype(jnp.float32)
                    cw = jnp.where(mask, jnp.where(jf < rf, qp1, qf), 0.0)
                    h = []
                    for i in range(HID):
                        hv = x * W1v[0, i] + y * W1v[1, i] + z * W1v[2, i]
                        h.append(jnp.maximum(hv + b1v[i], 0.0))

                    def do_out(o, _):
                        f = b2v[o]
                        for i in range(HID):
                            f = f + h[i] * W2v[i, o]
                        fm = jnp.where(mask, f, NEG)
                        amax[o] = jnp.maximum(amax[o], fm)
                        asum[o] = asum[o] + cw * f
                        return ()
                    lax.fori_loop(0, FEAT, do_out, ())
                    return ()
                lax.fori_loop(0, ngroups, do_group, ())

                def heads(o, res):
                    smax = jnp.max(amax[o])
                    ssum = jnp.sum(asum[o])
                    return res + smax * Whv[o] + ssum * Whv[FEAT + o]
                resv = lax.fori_loop(0, FEAT, heads, dfv[...])
                outv[k] = resv
            return ()

        for cc in range(NCHUNK):
            base = wid * PW + cc * CH
            pltpu.sync_copy(pts_h.at[pl.ds(base, CH)], pts_v)
            pltpu.sync_copy(ppre_h.at[pl.ds(base, CH)], ppre_v)
            pltpu.sync_copy(nlen_h.at[pl.ds(base, CH)], nlen_v)
            lax.fori_loop(0, CH, do_proposal, ())
            pltpu.sync_copy(outv, out_h.at[pl.ds(base, CH)])

    return body(pts_flat, ppre, nlen, W1b, b1b, W2b, b2b, Whead, defaults)


def _tc_call(pts128, c4, iv4, vb, W1x, b1x, W2p, b2p, Whead):
    PPR = 32
    NB1 = PPR * 128
    P = pts128.shape[0]
    MAXN = 256
    BP = 32
    R = BP * 256 // PPR
    M = BP * MAXN
    grid = (P // BP,)

    def body(pts_ref, c_ref, iv_ref, vb_ref, W1_ref, b1_ref, W2_ref, b2_ref,
             Wh_ref, out_ref):
        c4 = c_ref[...][:, :4]
        iv4 = iv_ref[...][:, :4]
        ctile = jnp.concatenate([c4] * PPR, axis=1)
        ivtile = jnp.concatenate([iv4] * PPR, axis=1)
        RPP = 256 // PPR
        cb = jnp.broadcast_to(ctile[:, None, :], (BP, RPP, 128)).reshape(R, 128)
        ivb = jnp.broadcast_to(ivtile[:, None, :], (BP, RPP, 128)).reshape(R, 128)
        x128 = (pts_ref[...].reshape(R, 128) - cb) * ivb
        hb = jnp.maximum(
            jnp.dot(x128, W1_ref[...], preferred_element_type=jnp.float32)
            + b1_ref[...][0][None, :], 0.0
        ).astype(jnp.bfloat16)
        f = (jnp.dot(hb.reshape(M, 128), W2_ref[...],
                     preferred_element_type=jnp.float32).astype(jnp.bfloat16)
             + b2_ref[...][0][None, :])
        cw3 = f[:, FEAT:FEAT + 1].reshape(BP, MAXN, 1)
        f3 = f[:, :FEAT].reshape(BP, MAXN, FEAT)
        fw = f3 * cw3
        fmax = jnp.max(f3, axis=1).astype(jnp.float32)
        fsum = jnp.sum(fw, axis=1, dtype=jnp.float32)
        valid = vb_ref[...][:, 0:1] > 0.0
        fmax = jnp.where(valid, fmax, 0.0)
        fsum = jnp.where(valid, fsum, 0.0)
        feats = jnp.concatenate(
            [fmax, fsum, jnp.ones((BP, 8), jnp.float32)], axis=-1)
        out_ref[...] = jnp.dot(feats, Wh_ref[...],
                               preferred_element_type=jnp.float32)

    return pl.pallas_call(
        body,
        grid=grid,
        in_specs=[
            pl.BlockSpec((BP, MAXN * 4), lambda i: (i, 0)),
            pl.BlockSpec((BP, 8), lambda i: (i, 0)),
            pl.BlockSpec((BP, 8), lambda i: (i, 0)),
            pl.BlockSpec((BP, 8), lambda i: (i, 0)),
            pl.BlockSpec((128, NB1), lambda i: (0, 0)),
            pl.BlockSpec((8, NB1), lambda i: (0, 0)),
            pl.BlockSpec((128, 128), lambda i: (0, 0)),
            pl.BlockSpec((8, 128), lambda i: (0, 0)),
            pl.BlockSpec((2 * FEAT + 8, 8), lambda i: (0, 0)),
        ],
        out_specs=pl.BlockSpec((BP, 8), lambda i: (i, 0)),
        out_shape=jax.ShapeDtypeStruct((P, 8), jnp.float32),
    )(pts128, c4, iv4, vb, W1x, b1x, W2p, b2p, Whead)


SC_ROWS = 512


def kernel(points, proposals, W1, b1, W2, b2, Wc, bc, Wr, br, lengths):
    P, MAXN, _ = points.shape
    n = lengths.astype(jnp.int32)
    safe = jnp.maximum(n, 1)
    q = MAXN // safe
    r = MAXN - q * safe
    center = proposals[:, :3]
    inv = 1.0 / (proposals[:, 3:6] + 1e-6)
    qf = q.astype(jnp.float32) / MAXN
    qp1 = qf + 1.0 / MAXN
    rf = r.astype(jnp.float32)

    PSC = SC_ROWS
    cls_parts, reg_parts = [], []

    if PSC > 0:
        ppre = jnp.zeros((PSC, L), jnp.float32)
        ppre = ppre.at[:, 0:3].set(center[:PSC]).at[:, 3:6].set(inv[:PSC])
        ppre = (ppre.at[:, 6].set(qf[:PSC]).at[:, 7].set(qp1[:PSC])
                .at[:, 8].set(rf[:PSC]))
        pts_flat = points[:PSC].reshape(PSC, MAXN * 3)
        W1b = jnp.broadcast_to(W1[:, :, None], (3, HID, L))
        b1b = jnp.broadcast_to(b1[:, None], (HID, L))
        W2b = jnp.broadcast_to(W2[:, :, None], (HID, FEAT, L))
        b2b = jnp.broadcast_to(b2[:, None], (FEAT, L))
        Whd = jnp.zeros((2 * FEAT, L), jnp.float32)
        Whd = Whd.at[:, 0].set(Wc[:, 0]).at[:, 1:5].set(Wr)
        defaults = jnp.zeros((L,), jnp.float32)
        defaults = defaults.at[0].set(bc[0]).at[1:5].set(br)
        out_sc = _sc_call(pts_flat, ppre, n[:PSC], W1b, b1b, W2b, b2b, Whd,
                          defaults)
        cls_parts.append(out_sc[:, :1])
        reg_parts.append(out_sc[:, 1:5])

    if PSC < P:
        jrow = jnp.arange(MAXN, dtype=jnp.int32)[None, :]
        cwtc = jnp.where(jrow < n[PSC:, None],
                         jnp.where(jrow < r[PSC:, None], qp1[PSC:, None],
                                   qf[PSC:, None]), 0.0)
        pts128 = jnp.concatenate(
            [points[PSC:], cwtc[:, :, None]], axis=-1
        ).reshape(P - PSC, MAXN * 4)
        c4 = jnp.zeros((P - PSC, 8), jnp.float32).at[:, :3].set(center[PSC:])
        iv4 = (jnp.zeros((P - PSC, 8), jnp.float32).at[:, :3].set(inv[PSC:])
               .at[:, 3].set(1.0))
        vb = jnp.zeros((P - PSC, 8), jnp.float32)
        vb = vb.at[:, 0].set((n[PSC:] >= MIN_N).astype(jnp.float32))
        W1blk = jnp.zeros((4, 128), jnp.float32).at[:3, :HID].set(W1)
        W1blk = W1blk.at[3, HID].set(-512.0).at[3, HID + 1].set(1.0)
        b1blk = jnp.zeros((128,), jnp.float32).at[:HID].set(b1)
        b1blk = b1blk.at[HID].set(1.0)
        W1x = jnp.kron(jnp.eye(32, dtype=jnp.float32), W1blk)
        b1x = jnp.broadcast_to(jnp.tile(b1blk, 32)[None, :], (8, 32 * 128))
        W2p = jnp.zeros((128, 128), jnp.float32).at[:HID, :FEAT].set(W2)
        W2p = W2p.at[HID, :FEAT].set(NEG).at[HID + 1, FEAT].set(1.0)
        W2p = W2p.astype(jnp.bfloat16)
        b2p = jnp.broadcast_to(
            jnp.zeros((128,), jnp.float32).at[:FEAT].set(b2)[None, :],
            (8, 128)).astype(jnp.bfloat16)
        Whead = jnp.zeros((2 * FEAT + 8, 8), jnp.float32)
        Whead = Whead.at[:FEAT, 0].set(Wc[:FEAT, 0]).at[:FEAT, 1:5].set(Wr[:FEAT])
        Whead = (Whead.at[FEAT:2 * FEAT, 0].set(Wc[FEAT:, 0])
                 .at[FEAT:2 * FEAT, 1:5].set(Wr[FEAT:]))
        Whead = Whead.at[2 * FEAT, 0].set(bc[0]).at[2 * FEAT, 1:5].set(br)
        out_tc = _tc_call(pts128, c4, iv4, vb, W1x, b1x, W2p, b2p, Whead)
        cls_parts.append(out_tc[:, :1])
        reg_parts.append(out_tc[:, 1:5])

    cls = jnp.concatenate(cls_parts, axis=0) if len(cls_parts) > 1 else cls_parts[0]
    reg = jnp.concatenate(reg_parts, axis=0) if len(reg_parts) > 1 else reg_parts[0]
    return cls, reg


# speedup vs baseline: 1.3515x; 1.0259x over previous
import functools

import jax
import jax.numpy as jnp
from jax import lax
from jax.experimental import pallas as pl
from jax.experimental.pallas import tpu as pltpu
from jax.experimental.pallas import tpu_sc as plsc

MIN_N = 4
HID = 32
FEAT = 64
L = 16
NEG = -3.0e38


def _sc_call(pts_flat, ppre, nlen, W1b, b1b, W2b, b2b, Whead, defaults):
    P = pts_flat.shape[0]
    NW = 32
    PW = P // NW
    CH = min(PW, 64)
    NCHUNK = PW // CH
    mesh = plsc.VectorSubcoreMesh(core_axis_name="c", subcore_axis_name="s")

    @functools.partial(
        pl.kernel,
        mesh=mesh,
        compiler_params=pltpu.CompilerParams(
            needs_layout_passes=False, use_tc_tiling_on_sc=False
        ),
        out_type=jax.ShapeDtypeStruct((P, L), jnp.float32),
        scratch_types=[
            pltpu.VMEM((CH, 3 * 256), jnp.float32),
            pltpu.VMEM((CH, L), jnp.float32),
            pltpu.VMEM((CH,), jnp.int32),
            pltpu.VMEM((3, HID, L), jnp.float32),
            pltpu.VMEM((HID, L), jnp.float32),
            pltpu.VMEM((HID, FEAT, L), jnp.float32),
            pltpu.VMEM((FEAT, L), jnp.float32),
            pltpu.VMEM((2 * FEAT, L), jnp.float32),
            pltpu.VMEM((L,), jnp.float32),
            pltpu.VMEM((FEAT, L), jnp.float32),
            pltpu.VMEM((FEAT, L), jnp.float32),
            pltpu.VMEM((CH, L), jnp.float32),
        ],
    )
    def body(pts_h, ppre_h, nlen_h, W1b_h, b1b_h, W2b_h, b2b_h, Wh_h, df_h,
             out_h, pts_v, ppre_v, nlen_v, W1v, b1v, W2v, b2v, Whv, dfv,
             amax, asum, outv):
        wid = lax.axis_index("s") * 2 + lax.axis_index("c")
        pltpu.sync_copy(W1b_h, W1v)
        pltpu.sync_copy(b1b_h, b1v)
        pltpu.sync_copy(W2b_h, W2v)
        pltpu.sync_copy(b2b_h, b2v)
        pltpu.sync_copy(Wh_h, Whv)
        pltpu.sync_copy(df_h, dfv)

        iota = lax.iota(jnp.int32, L)

        def do_proposal(k, _):
            ksp = jnp.full((L,), k, jnp.int32)
            nvec = plsc.load_gather(nlen_v, [ksp])
            n_s = jnp.max(nvec)

            def splat(d):
                return plsc.load_gather(ppre_v, [ksp, jnp.full((L,), d, jnp.int32)])

            cx, cy, cz = splat(0), splat(1), splat(2)
            ivx, ivy, ivz = splat(3), splat(4), splat(5)
            qf, qp1, rf = splat(6), splat(7), splat(8)

            outv[k] = dfv[...]

            @pl.when(n_s >= MIN_N)
            def _():
                def init_acc(o, _):
                    amax[o] = jnp.full((L,), NEG, jnp.float32)
                    asum[o] = jnp.zeros((L,), jnp.float32)
                    return ()
                lax.fori_loop(0, FEAT, init_acc, ())

                ngroups = lax.shift_right_logical(n_s + (L - 1), 4)

                def do_group(g, _):
                    jvec = g * L + iota
                    mask = jvec < nvec
                    jc = jnp.minimum(jvec, 255) * 3
                    x = plsc.load_gather(pts_v, [ksp, jc])
                    y = plsc.load_gather(pts_v, [ksp, jc + 1])
                    z = plsc.load_gather(pts_v, [ksp, jc + 2])
                    x = (x - cx) * ivx
                    y = (y - cy) * ivy
                    z = (z - cz) * ivz
                    jf = jvec.astype(jnp.float32)
                    cw = jnp.where(mask, jnp.where(jf < rf, qp1, qf), 0.0)
                    h = []
                    for i in range(HID):
                        hv = x * W1v[0, i] + y * W1v[1, i] + z * W1v[2, i]
                        h.append(jnp.maximum(hv + b1v[i], 0.0))

                    def do_out(o, _):
                        f = b2v[o]
                        for i in range(HID):
                            f = f + h[i] * W2v[i, o]
                        fm = jnp.where(mask, f, NEG)
                        amax[o] = jnp.maximum(amax[o], fm)
                        asum[o] = asum[o] + cw * f
                        return ()
                    lax.fori_loop(0, FEAT, do_out, ())
                    return ()
                lax.fori_loop(0, ngroups, do_group, ())

                def heads(o, res):
                    smax = jnp.max(amax[o])
                    ssum = jnp.sum(asum[o])
                    return res + smax * Whv[o] + ssum * Whv[FEAT + o]
                resv = lax.fori_loop(0, FEAT, heads, dfv[...])
                outv[k] = resv
            return ()

        for cc in range(NCHUNK):
            base = wid * PW + cc * CH
            pltpu.sync_copy(pts_h.at[pl.ds(base, CH)], pts_v)
            pltpu.sync_copy(ppre_h.at[pl.ds(base, CH)], ppre_v)
            pltpu.sync_copy(nlen_h.at[pl.ds(base, CH)], nlen_v)
            lax.fori_loop(0, CH, do_proposal, ())
            pltpu.sync_copy(outv, out_h.at[pl.ds(base, CH)])

    return body(pts_flat, ppre, nlen, W1b, b1b, W2b, b2b, Whead, defaults)


def _tc_call(pts128, c4, iv4, vb, W1x, b1x, W2p, b2p, Whead):
    PPR = 32
    NB1 = PPR * 128
    P = pts128.shape[0]
    MAXN = 256
    BP = 64
    R = BP * 256 // PPR
    M = BP * MAXN
    grid = (P // BP,)

    def body(pts_ref, c_ref, iv_ref, vb_ref, W1_ref, b1_ref, W2_ref, b2_ref,
             Wh_ref, out_ref):
        c4 = c_ref[...][:, :4]
        iv4 = iv_ref[...][:, :4]
        ctile = jnp.concatenate([c4] * PPR, axis=1)
        ivtile = jnp.concatenate([iv4] * PPR, axis=1)
        RPP = 256 // PPR
        cb = jnp.broadcast_to(ctile[:, None, :], (BP, RPP, 128)).reshape(R, 128)
        ivb = jnp.broadcast_to(ivtile[:, None, :], (BP, RPP, 128)).reshape(R, 128)
        x128 = (pts_ref[...].reshape(R, 128) - cb) * ivb
        hb = jnp.maximum(
            jnp.dot(x128, W1_ref[...], preferred_element_type=jnp.float32)
            + b1_ref[...][0][None, :], 0.0
        ).astype(jnp.bfloat16)
        f = (jnp.dot(hb.reshape(M, 128), W2_ref[...],
                     preferred_element_type=jnp.float32).astype(jnp.bfloat16)
             + b2_ref[...][0][None, :])
        cw3 = f[:, FEAT:FEAT + 1].reshape(BP, MAXN, 1)
        f3 = f[:, :FEAT].reshape(BP, MAXN, FEAT)
        fw = f3 * cw3
        fmax = jnp.max(f3, axis=1).astype(jnp.float32)
        fsum = jnp.sum(fw, axis=1, dtype=jnp.float32)
        valid = vb_ref[...][:, 0:1] > 0.0
        fmax = jnp.where(valid, fmax, 0.0)
        fsum = jnp.where(valid, fsum, 0.0)
        feats = jnp.concatenate(
            [fmax, fsum, jnp.ones((BP, 8), jnp.float32)], axis=-1)
        out_ref[...] = jnp.dot(feats, Wh_ref[...],
                               preferred_element_type=jnp.float32)

    return pl.pallas_call(
        body,
        grid=grid,
        in_specs=[
            pl.BlockSpec((BP, MAXN * 4), lambda i: (i, 0)),
            pl.BlockSpec((BP, 8), lambda i: (i, 0)),
            pl.BlockSpec((BP, 8), lambda i: (i, 0)),
            pl.BlockSpec((BP, 8), lambda i: (i, 0)),
            pl.BlockSpec((128, NB1), lambda i: (0, 0)),
            pl.BlockSpec((8, NB1), lambda i: (0, 0)),
            pl.BlockSpec((128, 128), lambda i: (0, 0)),
            pl.BlockSpec((8, 128), lambda i: (0, 0)),
            pl.BlockSpec((2 * FEAT + 8, 8), lambda i: (0, 0)),
        ],
        out_specs=pl.BlockSpec((BP, 8), lambda i: (i, 0)),
        out_shape=jax.ShapeDtypeStruct((P, 8), jnp.float32),
    )(pts128, c4, iv4, vb, W1x, b1x, W2p, b2p, Whead)


SC_ROWS = 512


def kernel(points, proposals, W1, b1, W2, b2, Wc, bc, Wr, br, lengths):
    P, MAXN, _ = points.shape
    n = lengths.astype(jnp.int32)
    safe = jnp.maximum(n, 1)
    q = MAXN // safe
    r = MAXN - q * safe
    center = proposals[:, :3]
    inv = 1.0 / (proposals[:, 3:6] + 1e-6)
    qf = q.astype(jnp.float32) / MAXN
    qp1 = qf + 1.0 / MAXN
    rf = r.astype(jnp.float32)

    PSC = SC_ROWS
    cls_parts, reg_parts = [], []

    if PSC > 0:
        ppre = jnp.zeros((PSC, L), jnp.float32)
        ppre = ppre.at[:, 0:3].set(center[:PSC]).at[:, 3:6].set(inv[:PSC])
        ppre = (ppre.at[:, 6].set(qf[:PSC]).at[:, 7].set(qp1[:PSC])
                .at[:, 8].set(rf[:PSC]))
        pts_flat = points[:PSC].reshape(PSC, MAXN * 3)
        W1b = jnp.broadcast_to(W1[:, :, None], (3, HID, L))
        b1b = jnp.broadcast_to(b1[:, None], (HID, L))
        W2b = jnp.broadcast_to(W2[:, :, None], (HID, FEAT, L))
        b2b = jnp.broadcast_to(b2[:, None], (FEAT, L))
        Whd = jnp.zeros((2 * FEAT, L), jnp.float32)
        Whd = Whd.at[:, 0].set(Wc[:, 0]).at[:, 1:5].set(Wr)
        defaults = jnp.zeros((L,), jnp.float32)
        defaults = defaults.at[0].set(bc[0]).at[1:5].set(br)
        out_sc = _sc_call(pts_flat, ppre, n[:PSC], W1b, b1b, W2b, b2b, Whd,
                          defaults)
        cls_parts.append(out_sc[:, :1])
        reg_parts.append(out_sc[:, 1:5])

    if PSC < P:
        jrow = jnp.arange(MAXN, dtype=jnp.int32)[None, :]
        cwtc = jnp.where(jrow < n[PSC:, None],
                         jnp.where(jrow < r[PSC:, None], qp1[PSC:, None],
                                   qf[PSC:, None]), 0.0)
        pts128 = jnp.concatenate(
            [points[PSC:], cwtc[:, :, None]], axis=-1
        ).reshape(P - PSC, MAXN * 4)
        c4 = jnp.zeros((P - PSC, 8), jnp.float32).at[:, :3].set(center[PSC:])
        iv4 = (jnp.zeros((P - PSC, 8), jnp.float32).at[:, :3].set(inv[PSC:])
               .at[:, 3].set(1.0))
        vb = jnp.zeros((P - PSC, 8), jnp.float32)
        vb = vb.at[:, 0].set((n[PSC:] >= MIN_N).astype(jnp.float32))
        W1blk = jnp.zeros((4, 128), jnp.float32).at[:3, :HID].set(W1)
        W1blk = W1blk.at[3, HID].set(-512.0).at[3, HID + 1].set(1.0)
        b1blk = jnp.zeros((128,), jnp.float32).at[:HID].set(b1)
        b1blk = b1blk.at[HID].set(1.0)
        W1x = jnp.kron(jnp.eye(32, dtype=jnp.float32), W1blk)
        b1x = jnp.broadcast_to(jnp.tile(b1blk, 32)[None, :], (8, 32 * 128))
        W2p = jnp.zeros((128, 128), jnp.float32).at[:HID, :FEAT].set(W2)
        W2p = W2p.at[HID, :FEAT].set(NEG).at[HID + 1, FEAT].set(1.0)
        W2p = W2p.astype(jnp.bfloat16)
        b2p = jnp.broadcast_to(
            jnp.zeros((128,), jnp.float32).at[:FEAT].set(b2)[None, :],
            (8, 128)).astype(jnp.bfloat16)
        Whead = jnp.zeros((2 * FEAT + 8, 8), jnp.float32)
        Whead = Whead.at[:FEAT, 0].set(Wc[:FEAT, 0]).at[:FEAT, 1:5].set(Wr[:FEAT])
        Whead = (Whead.at[FEAT:2 * FEAT, 0].set(Wc[FEAT:, 0])
                 .at[FEAT:2 * FEAT, 1:5].set(Wr[FEAT:]))
        Whead = Whead.at[2 * FEAT, 0].set(bc[0]).at[2 * FEAT, 1:5].set(br)
        out_tc = _tc_call(pts128, c4, iv4, vb, W1x, b1x, W2p, b2p, Whead)
        cls_parts.append(out_tc[:, :1])
        reg_parts.append(out_tc[:, 1:5])

    cls = jnp.concatenate(cls_parts, axis=0) if len(cls_parts) > 1 else cls_parts[0]
    reg = jnp.concatenate(reg_parts, axis=0) if len(reg_parts) > 1 else reg_parts[0]
    return cls, reg


# speedup vs baseline: 1.3888x; 1.0276x over previous
import functools

import jax
import jax.numpy as jnp
from jax import lax
from jax.experimental import pallas as pl
from jax.experimental.pallas import tpu as pltpu
from jax.experimental.pallas import tpu_sc as plsc

MIN_N = 4
HID = 32
FEAT = 64
L = 16
NEG = -3.0e38


def _sc_call(pts_flat, ppre, nlen, W1b, b1b, W2b, b2b, Whead, defaults):
    P = pts_flat.shape[0]
    NW = 32
    PW = P // NW
    CH = min(PW, 64)
    NCHUNK = PW // CH
    mesh = plsc.VectorSubcoreMesh(core_axis_name="c", subcore_axis_name="s")

    @functools.partial(
        pl.kernel,
        mesh=mesh,
        compiler_params=pltpu.CompilerParams(
            needs_layout_passes=False, use_tc_tiling_on_sc=False
        ),
        out_type=jax.ShapeDtypeStruct((P, L), jnp.float32),
        scratch_types=[
            pltpu.VMEM((CH, 3 * 256), jnp.float32),
            pltpu.VMEM((CH, L), jnp.float32),
            pltpu.VMEM((CH,), jnp.int32),
            pltpu.VMEM((3, HID, L), jnp.float32),
            pltpu.VMEM((HID, L), jnp.float32),
            pltpu.VMEM((HID, FEAT, L), jnp.float32),
            pltpu.VMEM((FEAT, L), jnp.float32),
            pltpu.VMEM((2 * FEAT, L), jnp.float32),
            pltpu.VMEM((L,), jnp.float32),
            pltpu.VMEM((FEAT, L), jnp.float32),
            pltpu.VMEM((FEAT, L), jnp.float32),
            pltpu.VMEM((CH, L), jnp.float32),
        ],
    )
    def body(pts_h, ppre_h, nlen_h, W1b_h, b1b_h, W2b_h, b2b_h, Wh_h, df_h,
             out_h, pts_v, ppre_v, nlen_v, W1v, b1v, W2v, b2v, Whv, dfv,
             amax, asum, outv):
        wid = lax.axis_index("s") * 2 + lax.axis_index("c")
        pltpu.sync_copy(W1b_h, W1v)
        pltpu.sync_copy(b1b_h, b1v)
        pltpu.sync_copy(W2b_h, W2v)
        pltpu.sync_copy(b2b_h, b2v)
        pltpu.sync_copy(Wh_h, Whv)
        pltpu.sync_copy(df_h, dfv)

        iota = lax.iota(jnp.int32, L)

        def do_proposal(k, _):
            ksp = jnp.full((L,), k, jnp.int32)
            nvec = plsc.load_gather(nlen_v, [ksp])
            n_s = jnp.max(nvec)

            def splat(d):
                return plsc.load_gather(ppre_v, [ksp, jnp.full((L,), d, jnp.int32)])

            cx, cy, cz = splat(0), splat(1), splat(2)
            ivx, ivy, ivz = splat(3), splat(4), splat(5)
            qf, qp1, rf = splat(6), splat(7), splat(8)

            outv[k] = dfv[...]

            @pl.when(n_s >= MIN_N)
            def _():
                def init_acc(o, _):
                    amax[o] = jnp.full((L,), NEG, jnp.float32)
                    asum[o] = jnp.zeros((L,), jnp.float32)
                    return ()
                lax.fori_loop(0, FEAT, init_acc, ())

                ngroups = lax.shift_right_logical(n_s + (L - 1), 4)

                def do_group(g, _):
                    jvec = g * L + iota
                    mask = jvec < nvec
                    jc = jnp.minimum(jvec, 255) * 3
                    x = plsc.load_gather(pts_v, [ksp, jc])
                    y = plsc.load_gather(pts_v, [ksp, jc + 1])
                    z = plsc.load_gather(pts_v, [ksp, jc + 2])
                    x = (x - cx) * ivx
                    y = (y - cy) * ivy
                    z = (z - cz) * ivz
                    jf = jvec.astype(jnp.float32)
                    cw = jnp.where(mask, jnp.where(jf < rf, qp1, qf), 0.0)
                    h = []
                    for i in range(HID):
                        hv = x * W1v[0, i] + y * W1v[1, i] + z * W1v[2, i]
                        h.append(jnp.maximum(hv + b1v[i], 0.0))

                    def do_out(o, _):
                        f = b2v[o]
                        for i in range(HID):
                            f = f + h[i] * W2v[i, o]
                        fm = jnp.where(mask, f, NEG)
                        amax[o] = jnp.maximum(amax[o], fm)
                        asum[o] = asum[o] + cw * f
                        return ()
                    lax.fori_loop(0, FEAT, do_out, ())
                    return ()
                lax.fori_loop(0, ngroups, do_group, ())

                def heads(o, res):
                    smax = jnp.max(amax[o])
                    ssum = jnp.sum(asum[o])
                    return res + smax * Whv[o] + ssum * Whv[FEAT + o]
                resv = lax.fori_loop(0, FEAT, heads, dfv[...])
                outv[k] = resv
            return ()

        for cc in range(NCHUNK):
            base = wid * PW + cc * CH
            pltpu.sync_copy(pts_h.at[pl.ds(base, CH)], pts_v)
            pltpu.sync_copy(ppre_h.at[pl.ds(base, CH)], ppre_v)
            pltpu.sync_copy(nlen_h.at[pl.ds(base, CH)], nlen_v)
            lax.fori_loop(0, CH, do_proposal, ())
            pltpu.sync_copy(outv, out_h.at[pl.ds(base, CH)])

    return body(pts_flat, ppre, nlen, W1b, b1b, W2b, b2b, Whead, defaults)


def _tc_call(pts128, c4, iv4, vb, W1x, b1x, W2p, b2p, Whead):
    PPR = 32
    NB1 = PPR * 128
    P = pts128.shape[0]
    MAXN = 256
    BP = 128
    R = BP * 256 // PPR
    M = BP * MAXN
    grid = (P // BP,)

    def body(pts_ref, c_ref, iv_ref, vb_ref, W1_ref, b1_ref, W2_ref, b2_ref,
             Wh_ref, out_ref):
        c4 = c_ref[...][:, :4]
        iv4 = iv_ref[...][:, :4]
        ctile = jnp.concatenate([c4] * PPR, axis=1)
        ivtile = jnp.concatenate([iv4] * PPR, axis=1)
        RPP = 256 // PPR
        cb = jnp.broadcast_to(ctile[:, None, :], (BP, RPP, 128)).reshape(R, 128)
        ivb = jnp.broadcast_to(ivtile[:, None, :], (BP, RPP, 128)).reshape(R, 128)
        x128 = (pts_ref[...].reshape(R, 128) - cb) * ivb
        hb = jnp.maximum(
            jnp.dot(x128, W1_ref[...], preferred_element_type=jnp.float32)
            + b1_ref[...][0][None, :], 0.0
        ).astype(jnp.bfloat16)
        f = (jnp.dot(hb.reshape(M, 128), W2_ref[...],
                     preferred_element_type=jnp.float32).astype(jnp.bfloat16)
             + b2_ref[...][0][None, :])
        cw3 = f[:, FEAT:FEAT + 1].reshape(BP, MAXN, 1)
        f3 = f[:, :FEAT].reshape(BP, MAXN, FEAT)
        fw = f3 * cw3
        fmax = jnp.max(f3, axis=1).astype(jnp.float32)
        fsum = jnp.sum(fw, axis=1, dtype=jnp.float32)
        valid = vb_ref[...][:, 0:1] > 0.0
        fmax = jnp.where(valid, fmax, 0.0)
        fsum = jnp.where(valid, fsum, 0.0)
        feats = jnp.concatenate(
            [fmax, fsum, jnp.ones((BP, 8), jnp.float32)], axis=-1)
        out_ref[...] = jnp.dot(feats, Wh_ref[...],
                               preferred_element_type=jnp.float32)

    return pl.pallas_call(
        body,
        grid=grid,
        in_specs=[
            pl.BlockSpec((BP, MAXN * 4), lambda i: (i, 0)),
            pl.BlockSpec((BP, 8), lambda i: (i, 0)),
            pl.BlockSpec((BP, 8), lambda i: (i, 0)),
            pl.BlockSpec((BP, 8), lambda i: (i, 0)),
            pl.BlockSpec((128, NB1), lambda i: (0, 0)),
            pl.BlockSpec((8, NB1), lambda i: (0, 0)),
            pl.BlockSpec((128, 128), lambda i: (0, 0)),
            pl.BlockSpec((8, 128), lambda i: (0, 0)),
            pl.BlockSpec((2 * FEAT + 8, 8), lambda i: (0, 0)),
        ],
        out_specs=pl.BlockSpec((BP, 8), lambda i: (i, 0)),
        out_shape=jax.ShapeDtypeStruct((P, 8), jnp.float32),
    )(pts128, c4, iv4, vb, W1x, b1x, W2p, b2p, Whead)


SC_ROWS = 512


def kernel(points, proposals, W1, b1, W2, b2, Wc, bc, Wr, br, lengths):
    P, MAXN, _ = points.shape
    n = lengths.astype(jnp.int32)
    safe = jnp.maximum(n, 1)
    q = MAXN // safe
    r = MAXN - q * safe
    center = proposals[:, :3]
    inv = 1.0 / (proposals[:, 3:6] + 1e-6)
    qf = q.astype(jnp.float32) / MAXN
    qp1 = qf + 1.0 / MAXN
    rf = r.astype(jnp.float32)

    PSC = SC_ROWS
    cls_parts, reg_parts = [], []

    if PSC > 0:
        ppre = jnp.zeros((PSC, L), jnp.float32)
        ppre = ppre.at[:, 0:3].set(center[:PSC]).at[:, 3:6].set(inv[:PSC])
        ppre = (ppre.at[:, 6].set(qf[:PSC]).at[:, 7].set(qp1[:PSC])
                .at[:, 8].set(rf[:PSC]))
        pts_flat = points[:PSC].reshape(PSC, MAXN * 3)
        W1b = jnp.broadcast_to(W1[:, :, None], (3, HID, L))
        b1b = jnp.broadcast_to(b1[:, None], (HID, L))
        W2b = jnp.broadcast_to(W2[:, :, None], (HID, FEAT, L))
        b2b = jnp.broadcast_to(b2[:, None], (FEAT, L))
        Whd = jnp.zeros((2 * FEAT, L), jnp.float32)
        Whd = Whd.at[:, 0].set(Wc[:, 0]).at[:, 1:5].set(Wr)
        defaults = jnp.zeros((L,), jnp.float32)
        defaults = defaults.at[0].set(bc[0]).at[1:5].set(br)
        out_sc = _sc_call(pts_flat, ppre, n[:PSC], W1b, b1b, W2b, b2b, Whd,
                          defaults)
        cls_parts.append(out_sc[:, :1])
        reg_parts.append(out_sc[:, 1:5])

    if PSC < P:
        jrow = jnp.arange(MAXN, dtype=jnp.int32)[None, :]
        cwtc = jnp.where(jrow < n[PSC:, None],
                         jnp.where(jrow < r[PSC:, None], qp1[PSC:, None],
                                   qf[PSC:, None]), 0.0)
        pts128 = jnp.concatenate(
            [points[PSC:], cwtc[:, :, None]], axis=-1
        ).reshape(P - PSC, MAXN * 4)
        c4 = jnp.zeros((P - PSC, 8), jnp.float32).at[:, :3].set(center[PSC:])
        iv4 = (jnp.zeros((P - PSC, 8), jnp.float32).at[:, :3].set(inv[PSC:])
               .at[:, 3].set(1.0))
        vb = jnp.zeros((P - PSC, 8), jnp.float32)
        vb = vb.at[:, 0].set((n[PSC:] >= MIN_N).astype(jnp.float32))
        W1blk = jnp.zeros((4, 128), jnp.float32).at[:3, :HID].set(W1)
        W1blk = W1blk.at[3, HID].set(-512.0).at[3, HID + 1].set(1.0)
        b1blk = jnp.zeros((128,), jnp.float32).at[:HID].set(b1)
        b1blk = b1blk.at[HID].set(1.0)
        W1x = jnp.kron(jnp.eye(32, dtype=jnp.float32), W1blk)
        b1x = jnp.broadcast_to(jnp.tile(b1blk, 32)[None, :], (8, 32 * 128))
        W2p = jnp.zeros((128, 128), jnp.float32).at[:HID, :FEAT].set(W2)
        W2p = W2p.at[HID, :FEAT].set(NEG).at[HID + 1, FEAT].set(1.0)
        W2p = W2p.astype(jnp.bfloat16)
        b2p = jnp.broadcast_to(
            jnp.zeros((128,), jnp.float32).at[:FEAT].set(b2)[None, :],
            (8, 128)).astype(jnp.bfloat16)
        Whead = jnp.zeros((2 * FEAT + 8, 8), jnp.float32)
        Whead = Whead.at[:FEAT, 0].set(Wc[:FEAT, 0]).at[:FEAT, 1:5].set(Wr[:FEAT])
        Whead = (Whead.at[FEAT:2 * FEAT, 0].set(Wc[FEAT:, 0])
                 .at[FEAT:2 * FEAT, 1:5].set(Wr[FEAT:]))
        Whead = Whead.at[2 * FEAT, 0].set(bc[0]).at[2 * FEAT, 1:5].set(br)
        out_tc = _tc_call(pts128, c4, iv4, vb, W1x, b1x, W2p, b2p, Whead)
        cls_parts.append(out_tc[:, :1])
        reg_parts.append(out_tc[:, 1:5])

    cls = jnp.concatenate(cls_parts, axis=0) if len(cls_parts) > 1 else cls_parts[0]
    reg = jnp.concatenate(reg_parts, axis=0) if len(reg_parts) > 1 else reg_parts[0]
    return cls, reg
